# Initial kernel scaffold; baseline (speedup 1.0000x reference)
import jax
import jax.numpy as jnp
from jax.experimental import pallas as pl


def kernel(x, edge_index, edge_weight, W1, b1, W2, b2):
    raise NotImplementedError("write your pallas kernel here")


# speedup vs baseline: 2.8328x; 2.8328x over previous
import functools

import jax
import jax.numpy as jnp
from jax import lax
from jax.experimental import pallas as pl
from jax.experimental.pallas import tpu as pltpu
from jax.experimental.pallas import tpu_sc as plsc

N = 10000
E = 320000
F_IN = 128
F_HID = 128
F_OUT = 40
F_OUT_PAD = 48

NC = 2
NS = 16
NW = NC * NS
E_T = E // NW
C = 80
NCH = E_T // C
R_T = 624
R_REM = N - NS * R_T


def _spmm_kernel_body(F, src_hbm, dst_hbm, w_hbm, feats_hbm, out_hbm,
                      src_v, dst_v, w_v, rows, acc, sem):
    nf = F // 16
    cid = lax.axis_index("c")
    sid = lax.axis_index("s")
    widg = cid * NS + sid

    zero = jnp.zeros((16,), jnp.float32)

    def zrow(r, _):
        for f in range(nf):
            rows[r, pl.ds(f * 16, 16)] = zero
        return _

    lax.fori_loop(0, C, zrow, 0, unroll=4)
    for k in range(R_T // C):
        pltpu.sync_copy(rows, acc.at[pl.ds(sid * R_T + k * C, C)])
    rem = R_T - (R_T // C) * C
    pltpu.sync_copy(rows.at[pl.ds(0, rem)],
                    acc.at[pl.ds(sid * R_T + R_T - rem, rem)])

    @pl.when(sid == NS - 1)
    def _():
        pltpu.sync_copy(rows.at[pl.ds(0, R_REM)],
                        acc.at[pl.ds(NS * R_T, R_REM)])

    plsc.subcore_barrier()

    def chunk(j, _):
        base = widg * E_T + j * C
        pltpu.sync_copy(src_hbm.at[pl.ds(base, C)], src_v)
        pltpu.sync_copy(dst_hbm.at[pl.ds(base, C)], dst_v)
        pltpu.sync_copy(w_hbm.at[widg, j], w_v)
        pltpu.async_copy(feats_hbm.at[src_v], rows, sem).wait()

        def edge(e, carry):
            wb = w_v[e, :]
            for f in range(nf):
                sl = (e, pl.ds(f * 16, 16))
                rows[sl] = rows[sl] * wb
            return carry

        lax.fori_loop(0, C, edge, 0, unroll=8)

        pltpu.sync_copy(rows, acc.at[dst_v], add=True)
        return _

    lax.fori_loop(0, NCH, chunk, 0)

    plsc.subcore_barrier()

    pltpu.sync_copy(acc.at[pl.ds(sid * R_T, R_T)],
                    out_hbm.at[cid, pl.ds(sid * R_T, R_T)])

    @pl.when(sid == NS - 1)
    def _():
        pltpu.sync_copy(acc.at[pl.ds(NS * R_T, R_REM)],
                        out_hbm.at[cid, pl.ds(NS * R_T, R_REM)])


@functools.lru_cache(maxsize=None)
def _make_spmm(F):
    mesh = plsc.VectorSubcoreMesh(core_axis_name="c", subcore_axis_name="s",
                                  num_cores=NC, num_subcores=NS)
    return pl.kernel(
        functools.partial(_spmm_kernel_body, F),
        mesh=mesh,
        out_type=jax.ShapeDtypeStruct((NC, N, F), jnp.float32),
        scratch_types=[
            pltpu.VMEM((C,), jnp.int32),
            pltpu.VMEM((C,), jnp.int32),
            pltpu.VMEM((C, 16), jnp.float32),
            pltpu.VMEM((C, F), jnp.float32),
            pltpu.VMEM_SHARED((N, F), jnp.float32),
            pltpu.SemaphoreType.DMA,
        ],
        compiler_params=pltpu.CompilerParams(use_tc_tiling_on_sc=False),
    )


_RB = 1000


def _mm_body(x_ref, w_ref, o_ref):
    o_ref[...] = jnp.dot(x_ref[...], w_ref[...],
                         preferred_element_type=jnp.float32)


def _fuse_body(p_ref, b_ref, w_ref, o_ref):
    h = jnp.maximum(p_ref[0] + p_ref[1] + b_ref[...], 0.0)
    o_ref[...] = jnp.dot(h, w_ref[...], preferred_element_type=jnp.float32)


def _comb_body(q_ref, b_ref, o_ref):
    s = q_ref[0] + q_ref[1]
    o_ref[...] = s[:, :F_OUT] + b_ref[...]


def _mm(x, w):
    return pl.pallas_call(
        _mm_body,
        grid=(N // _RB,),
        in_specs=[pl.BlockSpec((_RB, F_IN), lambda i: (i, 0)),
                  pl.BlockSpec((F_IN, F_HID), lambda i: (0, 0))],
        out_specs=pl.BlockSpec((_RB, F_HID), lambda i: (i, 0)),
        out_shape=jax.ShapeDtypeStruct((N, F_HID), jnp.float32),
    )(x, w)


def _fuse(p, b1, w2p):
    return pl.pallas_call(
        _fuse_body,
        grid=(N // _RB,),
        in_specs=[pl.BlockSpec((NC, _RB, F_HID), lambda i: (0, i, 0)),
                  pl.BlockSpec((1, F_HID), lambda i: (0, 0)),
                  pl.BlockSpec((F_HID, F_OUT_PAD), lambda i: (0, 0))],
        out_specs=pl.BlockSpec((_RB, F_OUT_PAD), lambda i: (i, 0)),
        out_shape=jax.ShapeDtypeStruct((N, F_OUT_PAD), jnp.float32),
    )(p, b1, w2p)


def _combine(q, b2):
    return pl.pallas_call(
        _comb_body,
        grid=(N // _RB,),
        in_specs=[pl.BlockSpec((NC, _RB, F_OUT_PAD), lambda i: (0, i, 0)),
                  pl.BlockSpec((1, F_OUT), lambda i: (0, 0))],
        out_specs=pl.BlockSpec((_RB, F_OUT), lambda i: (i, 0)),
        out_shape=jax.ShapeDtypeStruct((N, F_OUT), jnp.float32),
    )(q, b2)


def kernel(x, edge_index, edge_weight, W1, b1, W2, b2):
    src = edge_index[1].astype(jnp.int32)
    dst = edge_index[0].astype(jnp.int32)
    w = jnp.broadcast_to(
        edge_weight.astype(jnp.float32).reshape(NW, NCH, C, 1),
        (NW, NCH, C, 16))
    w2p = jnp.pad(W2, ((0, 0), (0, F_OUT_PAD - F_OUT)))

    xw1 = _mm(x, W1)
    p = _make_spmm(F_HID)(src, dst, w, xw1)
    hw2 = _fuse(p, b1.reshape(1, F_HID), w2p)
    q = _make_spmm(F_OUT_PAD)(src, dst, w, hw2)
    return _combine(q, b2.reshape(1, F_OUT))


# speedup vs baseline: 4.1396x; 1.4613x over previous
import functools

import jax
import jax.numpy as jnp
from jax import lax
from jax.experimental import pallas as pl
from jax.experimental.pallas import tpu as pltpu
from jax.experimental.pallas import tpu_sc as plsc

N = 10000
E = 320000
F_IN = 128
F_HID = 128
F_OUT = 40
F_OUT_PAD = 48

NC = 2
NS = 16
NW = NC * NS
E_T = E // NW
C = 80
NCH = E_T // C
R_T = 624
R_REM = N - NS * R_T


def _spmm_kernel_body(F, idx_hbm, w_hbm, feats_hbm, out_hbm,
                      idx_v0, idx_v1, w_v0, w_v1, rows0, rows1,
                      gsem0, gsem1, stsem0, stsem1, acc):
    nf = F // 16
    cid = lax.axis_index("c")
    sid = lax.axis_index("s")
    widg = cid * NS + sid
    bufs = ((idx_v0, w_v0, rows0, gsem0, stsem0),
            (idx_v1, w_v1, rows1, gsem1, stsem1))

    zero = jnp.zeros((16,), jnp.float32)

    def zrow(r, _):
        for f in range(nf):
            rows0[r, pl.ds(f * 16, 16)] = zero
        return _

    lax.fori_loop(0, C, zrow, 0, unroll=4)
    for k in range(R_T // C):
        pltpu.sync_copy(rows0, acc.at[pl.ds(sid * R_T + k * C, C)])
    rem = R_T - (R_T // C) * C
    pltpu.sync_copy(rows0.at[pl.ds(0, rem)],
                    acc.at[pl.ds(sid * R_T + R_T - rem, rem)])

    @pl.when(sid == NS - 1)
    def _():
        pltpu.sync_copy(rows0.at[pl.ds(0, R_REM)],
                        acc.at[pl.ds(NS * R_T, R_REM)])

    plsc.subcore_barrier()

    def stage_start(j, b):
        idx_v, w_v, _, _, stsem = bufs[b]
        pltpu.async_copy(idx_hbm.at[widg, j], idx_v, stsem)
        pltpu.async_copy(w_hbm.at[widg, j], w_v, stsem)

    def stage_wait(b):
        idx_v, w_v, _, _, stsem = bufs[b]
        pltpu.make_async_copy(idx_hbm.at[widg, 0], idx_v, stsem).wait()
        pltpu.make_async_copy(w_hbm.at[widg, 0], w_v, stsem).wait()

    def gather_start(b):
        idx_v, _, rows, gsem, _ = bufs[b]
        pltpu.async_copy(feats_hbm.at[idx_v.at[0]], rows, gsem)

    def gather_wait(b):
        idx_v, _, rows, gsem, _ = bufs[b]
        pltpu.make_async_copy(feats_hbm.at[idx_v.at[0]], rows, gsem).wait()

    def compute_scatter(b):
        idx_v, w_v, rows, _, _ = bufs[b]

        def edge(e, carry):
            wb = w_v[e, :]
            for f in range(nf):
                sl = (e, pl.ds(f * 16, 16))
                rows[sl] = rows[sl] * wb
            return carry

        lax.fori_loop(0, C, edge, 0, unroll=8)
        pltpu.sync_copy(rows, acc.at[idx_v.at[1]], add=True)

    stage_start(0, 0)
    stage_start(1, 1)
    stage_wait(0)
    gather_start(0)

    def pair(g, carry):
        j0 = 2 * g
        stage_wait(1)
        gather_start(1)
        gather_wait(0)
        compute_scatter(0)
        stage_start(j0 + 2, 0)
        stage_wait(0)
        gather_start(0)
        gather_wait(1)
        compute_scatter(1)

        @pl.when(j0 + 3 < NCH)
        def _stage_next():
            stage_start(j0 + 3, 1)

        return carry

    lax.fori_loop(0, NCH // 2, pair, 0)

    gather_wait(0)
    compute_scatter(0)

    plsc.subcore_barrier()

    pltpu.sync_copy(acc.at[pl.ds(sid * R_T, R_T)],
                    out_hbm.at[cid, pl.ds(sid * R_T, R_T)])

    @pl.when(sid == NS - 1)
    def _():
        pltpu.sync_copy(acc.at[pl.ds(NS * R_T, R_REM)],
                        out_hbm.at[cid, pl.ds(NS * R_T, R_REM)])


@functools.lru_cache(maxsize=None)
def _make_spmm(F):
    mesh = plsc.VectorSubcoreMesh(core_axis_name="c", subcore_axis_name="s",
                                  num_cores=NC, num_subcores=NS)
    return pl.kernel(
        functools.partial(_spmm_kernel_body, F),
        mesh=mesh,
        out_type=jax.ShapeDtypeStruct((NC, N, F), jnp.float32),
        scratch_types=[
            pltpu.VMEM((2, C), jnp.int32),
            pltpu.VMEM((2, C), jnp.int32),
            pltpu.VMEM((C, 16), jnp.float32),
            pltpu.VMEM((C, 16), jnp.float32),
            pltpu.VMEM((C, F), jnp.float32),
            pltpu.VMEM((C, F), jnp.float32),
            pltpu.SemaphoreType.DMA,
            pltpu.SemaphoreType.DMA,
            pltpu.SemaphoreType.DMA,
            pltpu.SemaphoreType.DMA,
            pltpu.VMEM_SHARED((N, F), jnp.float32),
        ],
        compiler_params=pltpu.CompilerParams(use_tc_tiling_on_sc=False),
    )


_RB = 1000


def _mm_body(x_ref, w_ref, o_ref):
    o_ref[...] = jnp.dot(x_ref[...], w_ref[...],
                         preferred_element_type=jnp.float32)


def _fuse_body(p_ref, b_ref, w_ref, o_ref):
    h = jnp.maximum(p_ref[0] + p_ref[1] + b_ref[...], 0.0)
    o_ref[...] = jnp.dot(h, w_ref[...], preferred_element_type=jnp.float32)


def _comb_body(q_ref, b_ref, o_ref):
    s = q_ref[0] + q_ref[1]
    o_ref[...] = s[:, :F_OUT] + b_ref[...]


def _mm(x, w):
    return pl.pallas_call(
        _mm_body,
        grid=(N // _RB,),
        in_specs=[pl.BlockSpec((_RB, F_IN), lambda i: (i, 0)),
                  pl.BlockSpec((F_IN, F_HID), lambda i: (0, 0))],
        out_specs=pl.BlockSpec((_RB, F_HID), lambda i: (i, 0)),
        out_shape=jax.ShapeDtypeStruct((N, F_HID), jnp.float32),
    )(x, w)


def _fuse(p, b1, w2p):
    return pl.pallas_call(
        _fuse_body,
        grid=(N // _RB,),
        in_specs=[pl.BlockSpec((NC, _RB, F_HID), lambda i: (0, i, 0)),
                  pl.BlockSpec((1, F_HID), lambda i: (0, 0)),
                  pl.BlockSpec((F_HID, F_OUT_PAD), lambda i: (0, 0))],
        out_specs=pl.BlockSpec((_RB, F_OUT_PAD), lambda i: (i, 0)),
        out_shape=jax.ShapeDtypeStruct((N, F_OUT_PAD), jnp.float32),
    )(p, b1, w2p)


def _combine(q, b2):
    return pl.pallas_call(
        _comb_body,
        grid=(N // _RB,),
        in_specs=[pl.BlockSpec((NC, _RB, F_OUT_PAD), lambda i: (0, i, 0)),
                  pl.BlockSpec((1, F_OUT), lambda i: (0, 0))],
        out_specs=pl.BlockSpec((_RB, F_OUT), lambda i: (i, 0)),
        out_shape=jax.ShapeDtypeStruct((N, F_OUT), jnp.float32),
    )(q, b2)


def kernel(x, edge_index, edge_weight, W1, b1, W2, b2):
    src = edge_index[1].astype(jnp.int32).reshape(NW, NCH, 1, C)
    dst = edge_index[0].astype(jnp.int32).reshape(NW, NCH, 1, C)
    idx = jnp.concatenate([src, dst], axis=2)
    w = jnp.broadcast_to(
        edge_weight.astype(jnp.float32).reshape(NW, NCH, C, 1),
        (NW, NCH, C, 16))
    w2p = jnp.pad(W2, ((0, 0), (0, F_OUT_PAD - F_OUT)))

    xw1 = _mm(x, W1)
    p = _make_spmm(F_HID)(idx, w, xw1)
    hw2 = _fuse(p, b1.reshape(1, F_HID), w2p)
    q = _make_spmm(F_OUT_PAD)(idx, w, hw2)
    return _combine(q, b2.reshape(1, F_OUT))
